# Initial kernel scaffold; baseline (speedup 1.0000x reference)
import jax
import jax.numpy as jnp
from jax.experimental import pallas as pl


def kernel(x, res, out, W_mlp):
    raise NotImplementedError("write your pallas kernel here")


# speedup vs baseline: 2.2223x; 2.2223x over previous
import functools

import jax
import jax.numpy as jnp
from jax import lax
from jax.experimental import pallas as pl
from jax.experimental.pallas import tpu as pltpu
from jax.experimental.pallas import tpu_sc as plsc

_EMBED = 96
_NC = 3
_KS = 3
_BETA = 0.75

_SC_CORES = 2
_SC_SUBCORES = 16
_NW = _SC_CORES * _SC_SUBCORES
_L = 16


def _dense_body(w_ref, x_ref, o_ref, r_ref, s0_ref, s1_ref, y_ref):
    o = o_ref[0]
    d = o - r_ref[0]
    d = d * d
    d0, d1, d2 = d[0:1], d[1:2], d[2:3]
    hi = jnp.maximum(d0, d1)
    lo = jnp.minimum(d0, d1)
    s0 = jnp.maximum(hi, d2)
    s1 = jnp.minimum(hi, jnp.maximum(lo, d2))
    s0_ref[0] = s0
    s1_ref[0] = s1
    w = w_ref[...]
    f = x_ref[0]
    y = lax.dot_general(w[:, :_NC], o, (((1,), (0,)), ((), ())),
                        preferred_element_type=jnp.float32)
    y = y + lax.dot_general(w[:, _NC:], f, (((1,), (0,)), ((), ())),
                            preferred_element_type=jnp.float32)
    y_ref[0] = y


def _dense_maps(x2, o2, r2, w_mlp, B, HW):
    T = 2048
    grid = (B, HW // T)
    return pl.pallas_call(
        _dense_body,
        grid=grid,
        in_specs=[
            pl.BlockSpec((_NC, _EMBED + _NC), lambda b, t: (0, 0)),
            pl.BlockSpec((1, _EMBED, T), lambda b, t: (b, 0, t)),
            pl.BlockSpec((1, _NC, T), lambda b, t: (b, 0, t)),
            pl.BlockSpec((1, _NC, T), lambda b, t: (b, 0, t)),
        ],
        out_specs=[
            pl.BlockSpec((1, 1, T), lambda b, t: (b, 0, t)),
            pl.BlockSpec((1, 1, T), lambda b, t: (b, 0, t)),
            pl.BlockSpec((1, _NC, T), lambda b, t: (b, 0, t)),
        ],
        out_shape=[
            jax.ShapeDtypeStruct((B, 1, HW), jnp.float32),
            jax.ShapeDtypeStruct((B, 1, HW), jnp.float32),
            jax.ShapeDtypeStruct((B, _NC, HW), jnp.float32),
        ],
    )(w_mlp, x2, o2, r2)


def _grid_coords(px, py, H, W):
    gridx = 2.0 * px - 1.0
    gridy = 2.0 * py - 1.0
    gx = ((gridx + 1.0) * W - 1.0) / 2.0
    gy = ((gridy + 1.0) * H - 1.0) / 2.0

    def _floor(g):
        ti = g.astype(jnp.int32)
        tf = ti.astype(jnp.float32)
        adj = g < tf
        tf = jnp.where(adj, tf - 1.0, tf)
        ti = jnp.where(adj, ti - 1, ti)
        return tf, ti

    x0f, x0i = _floor(gx)
    y0f, y0i = _floor(gy)
    wx1 = gx - x0f
    wx0 = 1.0 - wx1
    wy1 = gy - y0f
    wy0 = 1.0 - wy1
    return x0f, x0i, y0f, y0i, wx0, wx1, wy0, wy1


def _corner(xf, yf, xi, yi, H, W):
    valid = (xf >= 0.0) & (xf <= W - 1.0) & (yf >= 0.0) & (yf <= H - 1.0)
    vm = jnp.where(valid, 1.0, 0.0)
    ix = jnp.clip(xi, 0, W - 1)
    iy = jnp.clip(yi, 0, H - 1)
    return iy * W + ix, vm


def _make_u_kernel(B, H, W, KN):
    HW = H * W
    total = B * KN
    CH = 32
    nchunks = total // CH
    per_tile = nchunks // _NW
    mesh = plsc.VectorSubcoreMesh(core_axis_name="c", subcore_axis_name="s")

    @functools.partial(
        pl.kernel, mesh=mesh,
        out_type=jax.ShapeDtypeStruct((total,), jnp.float32),
        scratch_types=[
            pltpu.VMEM((CH,), jnp.float32),
            pltpu.VMEM((CH,), jnp.float32),
            pltpu.VMEM((16, _L), jnp.float32),
            pltpu.VMEM((CH,), jnp.float32),
            pltpu.SemaphoreType.DMA,
            pltpu.SemaphoreType.DMA,
        ],
    )
    def u_kernel(overx_hbm, overy_hbm, s0_hbm, s1_hbm, u_hbm,
                 cbufx, cbufy, gbuf, ubuf, sem, sem2):
        wid = lax.axis_index("s") * _SC_CORES + lax.axis_index("c")

        def chunk(i, _):
            cid = wid * per_tile + i
            base = cid * CH
            b = base // KN
            pltpu.sync_copy(overx_hbm.at[pl.ds(base, CH)], cbufx)
            pltpu.sync_copy(overy_hbm.at[pl.ds(base, CH)], cbufy)
            splane = [s0_hbm, s1_hbm]
            copies = []
            geom = []
            for j in range(CH // _L):
                px = cbufx[pl.ds(j * _L, _L)]
                py = cbufy[pl.ds(j * _L, _L)]
                x0f, x0i, y0f, y0i, wx0, wx1, wy0, wy1 = _grid_coords(
                    px, py, H, W)
                corners = [
                    (x0f, y0f, x0i, y0i),
                    (x0f + 1.0, y0f, x0i + 1, y0i),
                    (x0f, y0f + 1.0, x0i, y0i + 1),
                    (x0f + 1.0, y0f + 1.0, x0i + 1, y0i + 1),
                ]
                ws = [wy0 * wx0, wy0 * wx1, wy1 * wx0, wy1 * wx1]
                vms = []
                for q, (xf, yf, xi, yi) in enumerate(corners):
                    lin, vm = _corner(xf, yf, xi, yi, H, W)
                    gidx = b * HW + lin
                    vms.append(vm)
                    for p in range(2):
                        slot = 8 * j + 2 * q + p
                        copies.append(pltpu.async_copy(
                            splane[p].at[gidx], gbuf.at[slot],
                            sem if p == 0 else sem2))
                geom.append((ws, vms))
            for cp in copies:
                cp.wait()
            for j in range(CH // _L):
                ws, vms = geom[j]
                og = []
                for p in range(2):
                    v = [gbuf[8 * j + 2 * q + p] * vms[q] for q in range(4)]
                    og.append(((v[0] * ws[0] + v[1] * ws[1]) + v[2] * ws[2])
                              + v[3] * ws[3])
                u = -1.0 * (og[0] - og[1])
                ubuf[pl.ds(j * _L, _L)] = u
            pltpu.sync_copy(ubuf, u_hbm.at[pl.ds(base, CH)])
            return 0

        lax.fori_loop(0, per_tile, chunk, 0)

    return u_kernel


def _make_render_kernel(B, H, W, KN, N, NB):
    HW = H * W
    NCOV = N - NB
    imp_chunks = (B * NB) // _L
    cov_chunks = (B * NCOV) // _L
    imp_per_tile = imp_chunks // _NW
    cov_per_tile = cov_chunks // _NW
    mesh = plsc.VectorSubcoreMesh(core_axis_name="c", subcore_axis_name="s")

    @functools.partial(
        pl.kernel, mesh=mesh,
        out_type=(
            jax.ShapeDtypeStruct((B * NB,), jnp.float32),
            jax.ShapeDtypeStruct((B * NB,), jnp.float32),
            jax.ShapeDtypeStruct((B * _NC * N,), jnp.float32),
        ),
        scratch_types=[
            pltpu.VMEM((_L,), jnp.int32),
            pltpu.VMEM((_L,), jnp.float32),
            pltpu.VMEM((_L,), jnp.float32),
            pltpu.VMEM((12, _L), jnp.float32),
            pltpu.VMEM((3, _L), jnp.float32),
            pltpu.SemaphoreType.DMA,
            pltpu.SemaphoreType.DMA,
        ],
    )
    def render_kernel(idx_hbm, ox_hbm, oy_hbm, cx_hbm, cy_hbm, y_hbm,
                      ptsx_hbm, ptsy_hbm, rend_hbm,
                      ibuf, pxbuf, pybuf, gbuf, rbuf, sem, sem2):
        wid = lax.axis_index("s") * _SC_CORES + lax.axis_index("c")

        def sample(b, tglob):
            px = pxbuf[...]
            py = pybuf[...]
            x0f, x0i, y0f, y0i, wx0, wx1, wy0, wy1 = _grid_coords(
                px, py, H, W)
            corners = [
                (x0f, y0f, x0i, y0i),
                (x0f + 1.0, y0f, x0i + 1, y0i),
                (x0f, y0f + 1.0, x0i, y0i + 1),
                (x0f + 1.0, y0f + 1.0, x0i + 1, y0i + 1),
            ]
            ws = [wy0 * wx0, wy0 * wx1, wy1 * wx0, wy1 * wx1]
            copies = []
            vms = []
            for q, (xf, yf, xi, yi) in enumerate(corners):
                lin, vm = _corner(xf, yf, xi, yi, H, W)
                vms.append(vm)
                for ch in range(3):
                    gidx = (b * 3 + ch) * HW + lin
                    copies.append(pltpu.async_copy(
                        y_hbm.at[gidx], gbuf.at[3 * q + ch],
                        sem if ch != 1 else sem2))
            for cp in copies:
                cp.wait()
            for ch in range(3):
                v = [gbuf[3 * q + ch] * vms[q] for q in range(4)]
                r = ((v[0] * ws[0] + v[1] * ws[1]) + v[2] * ws[2]) \
                    + v[3] * ws[3]
                rbuf[ch] = jnp.maximum(r, 0.0)
            for ch in range(3):
                pltpu.sync_copy(
                    rbuf.at[ch],
                    rend_hbm.at[pl.ds((b * 3 + ch) * N + tglob, _L)])
            return 0

        def imp_chunk(i, _):
            cid = wid * imp_per_tile + i
            base = cid * _L
            b = base // NB
            t = base - b * NB
            pltpu.sync_copy(idx_hbm.at[pl.ds(base, _L)], ibuf)
            iv = ibuf[...] + b * KN
            cpx = pltpu.async_copy(ox_hbm.at[iv], pxbuf, sem)
            cpy = pltpu.async_copy(oy_hbm.at[iv], pybuf, sem2)
            cpx.wait()
            cpy.wait()
            pltpu.sync_copy(pxbuf, ptsx_hbm.at[pl.ds(base, _L)])
            pltpu.sync_copy(pybuf, ptsy_hbm.at[pl.ds(base, _L)])
            return sample(b, t)

        def cov_chunk(i, _):
            cid = wid * cov_per_tile + i
            base = cid * _L
            b = base // NCOV
            t = base - b * NCOV
            pltpu.sync_copy(cx_hbm.at[pl.ds(base, _L)], pxbuf)
            pltpu.sync_copy(cy_hbm.at[pl.ds(base, _L)], pybuf)
            return sample(b, NB + t)

        lax.fori_loop(0, imp_per_tile, imp_chunk, 0)
        lax.fori_loop(0, cov_per_tile, cov_chunk, 0)

    return render_kernel


def kernel(x, res, out, W_mlp):
    B, C, H, W = out.shape
    HW = H * W
    N = HW // 16
    KN = _KS * N
    NB = int(_BETA * N)

    key = jax.random.key(42)
    k1, k2 = jax.random.split(key)
    over = jax.random.uniform(k1, (B, KN, 2), dtype=x.dtype)
    coverage = jax.random.uniform(k2, (B, N - NB, 2), dtype=x.dtype)

    x2 = x.reshape(B, _EMBED, HW)
    o2 = out.reshape(B, C, HW)
    r2 = res.reshape(B, C, HW)
    s0, s1, y = _dense_maps(x2, o2, r2, W_mlp, B, HW)

    ox = over[:, :, 0].reshape(B * KN)
    oy = over[:, :, 1].reshape(B * KN)
    cx = coverage[:, :, 0].reshape(B * (N - NB))
    cy = coverage[:, :, 1].reshape(B * (N - NB))

    u_kernel = _make_u_kernel(B, H, W, KN)
    u = u_kernel(ox, oy, s0.reshape(B * HW), s1.reshape(B * HW))
    u = u.reshape(B, KN)

    _, idx = lax.top_k(u, NB)

    render = _make_render_kernel(B, H, W, KN, N, NB)
    ptsx, ptsy, rend = render(
        idx.reshape(B * NB).astype(jnp.int32),
        ox, oy, cx, cy,
        y.reshape(B * C * HW),
    )
    pts_imp = jnp.stack([ptsx, ptsy], axis=-1).reshape(B, NB, 2)
    points = jnp.concatenate([pts_imp, coverage], axis=1)
    rend = rend.reshape(B, C, N)
    return rend, points


# speedup vs baseline: 2.5908x; 1.1658x over previous
import functools

import jax
import jax.numpy as jnp
from jax import lax
from jax.experimental import pallas as pl
from jax.experimental.pallas import tpu as pltpu
from jax.experimental.pallas import tpu_sc as plsc

_EMBED = 96
_NC = 3
_KS = 3
_BETA = 0.75

_SC_CORES = 2
_SC_SUBCORES = 16
_NW = _SC_CORES * _SC_SUBCORES
_L = 16


def _smap_body(o_ref, r_ref, s0_ref, s1_ref):
    o = o_ref[0]
    d = o - r_ref[0]
    d = d * d
    d0, d1, d2 = d[0:1], d[1:2], d[2:3]
    hi = jnp.maximum(d0, d1)
    lo = jnp.minimum(d0, d1)
    s0 = jnp.maximum(hi, d2)
    s1 = jnp.minimum(hi, jnp.maximum(lo, d2))
    s0_ref[0] = s0
    s1_ref[0] = s1


def _smap_maps(o2, r2, B, HW):
    T = 4096
    grid = (B, HW // T)
    return pl.pallas_call(
        _smap_body,
        grid=grid,
        in_specs=[
            pl.BlockSpec((1, _NC, T), lambda b, t: (b, 0, t)),
            pl.BlockSpec((1, _NC, T), lambda b, t: (b, 0, t)),
        ],
        out_specs=[
            pl.BlockSpec((1, 1, T), lambda b, t: (b, 0, t)),
            pl.BlockSpec((1, 1, T), lambda b, t: (b, 0, t)),
        ],
        out_shape=[
            jax.ShapeDtypeStruct((B, 1, HW), jnp.float32),
            jax.ShapeDtypeStruct((B, 1, HW), jnp.float32),
        ],
    )(o2, r2)


def _ymap_body(w_ref, x_ref, o_ref, y_ref):
    o = o_ref[0]
    w = w_ref[...]
    f = x_ref[0]
    y = lax.dot_general(w[:, :_NC], o, (((1,), (0,)), ((), ())),
                        preferred_element_type=jnp.float32)
    y = y + lax.dot_general(w[:, _NC:], f, (((1,), (0,)), ((), ())),
                            preferred_element_type=jnp.float32)
    y_ref[0] = y


def _ymap_map(x2, o2, w_mlp, B, HW):
    T = 2048
    grid = (B, HW // T)
    return pl.pallas_call(
        _ymap_body,
        grid=grid,
        in_specs=[
            pl.BlockSpec((_NC, _EMBED + _NC), lambda b, t: (0, 0)),
            pl.BlockSpec((1, _EMBED, T), lambda b, t: (b, 0, t)),
            pl.BlockSpec((1, _NC, T), lambda b, t: (b, 0, t)),
        ],
        out_specs=pl.BlockSpec((1, _NC, T), lambda b, t: (b, 0, t)),
        out_shape=jax.ShapeDtypeStruct((B, _NC, HW), jnp.float32),
    )(w_mlp, x2, o2)


def _grid_coords(px, py, H, W):
    gridx = 2.0 * px - 1.0
    gridy = 2.0 * py - 1.0
    gx = ((gridx + 1.0) * W - 1.0) / 2.0
    gy = ((gridy + 1.0) * H - 1.0) / 2.0

    def _floor(g):
        ti = g.astype(jnp.int32)
        tf = ti.astype(jnp.float32)
        adj = g < tf
        tf = jnp.where(adj, tf - 1.0, tf)
        ti = jnp.where(adj, ti - 1, ti)
        return tf, ti

    x0f, x0i = _floor(gx)
    y0f, y0i = _floor(gy)
    wx1 = gx - x0f
    wx0 = 1.0 - wx1
    wy1 = gy - y0f
    wy0 = 1.0 - wy1
    return x0f, x0i, y0f, y0i, wx0, wx1, wy0, wy1


def _corner(xf, yf, xi, yi, H, W):
    valid = (xf >= 0.0) & (xf <= W - 1.0) & (yf >= 0.0) & (yf <= H - 1.0)
    vm = jnp.where(valid, 1.0, 0.0)
    ix = jnp.clip(xi, 0, W - 1)
    iy = jnp.clip(yi, 0, H - 1)
    return iy * W + ix, vm


def _make_u_kernel(B, H, W, KN):
    HW = H * W
    total = B * KN
    CH = 48
    nchunks = total // CH
    per_tile = nchunks // _NW
    mesh = plsc.VectorSubcoreMesh(core_axis_name="c", subcore_axis_name="s")

    @functools.partial(
        pl.kernel, mesh=mesh,
        out_type=jax.ShapeDtypeStruct((total,), jnp.float32),
        scratch_types=[
            pltpu.VMEM((CH,), jnp.float32),
            pltpu.VMEM((CH,), jnp.float32),
            pltpu.VMEM((24, _L), jnp.float32),
            pltpu.VMEM((CH,), jnp.float32),
            pltpu.SemaphoreType.DMA,
            pltpu.SemaphoreType.DMA,
        ],
    )
    def u_kernel(overx_hbm, overy_hbm, s0_hbm, s1_hbm, u_hbm,
                 cbufx, cbufy, gbuf, ubuf, sem, sem2):
        wid = lax.axis_index("s") * _SC_CORES + lax.axis_index("c")

        def chunk(i, _):
            cid = wid * per_tile + i
            base = cid * CH
            b = base // KN
            pltpu.sync_copy(overx_hbm.at[pl.ds(base, CH)], cbufx)
            pltpu.sync_copy(overy_hbm.at[pl.ds(base, CH)], cbufy)
            splane = [s0_hbm, s1_hbm]
            copies = []
            geom = []
            for j in range(CH // _L):
                px = cbufx[pl.ds(j * _L, _L)]
                py = cbufy[pl.ds(j * _L, _L)]
                x0f, x0i, y0f, y0i, wx0, wx1, wy0, wy1 = _grid_coords(
                    px, py, H, W)
                corners = [
                    (x0f, y0f, x0i, y0i),
                    (x0f + 1.0, y0f, x0i + 1, y0i),
                    (x0f, y0f + 1.0, x0i, y0i + 1),
                    (x0f + 1.0, y0f + 1.0, x0i + 1, y0i + 1),
                ]
                ws = [wy0 * wx0, wy0 * wx1, wy1 * wx0, wy1 * wx1]
                vms = []
                for q, (xf, yf, xi, yi) in enumerate(corners):
                    lin, vm = _corner(xf, yf, xi, yi, H, W)
                    gidx = b * HW + lin
                    vms.append(vm)
                    for p in range(2):
                        slot = 8 * j + 2 * q + p
                        copies.append(pltpu.async_copy(
                            splane[p].at[gidx], gbuf.at[slot],
                            sem if p == 0 else sem2))
                geom.append((ws, vms))
            for cp in copies:
                cp.wait()
            for j in range(CH // _L):
                ws, vms = geom[j]
                og = []
                for p in range(2):
                    v = [gbuf[8 * j + 2 * q + p] * vms[q] for q in range(4)]
                    og.append(((v[0] * ws[0] + v[1] * ws[1]) + v[2] * ws[2])
                              + v[3] * ws[3])
                u = -1.0 * (og[0] - og[1])
                ubuf[pl.ds(j * _L, _L)] = u
            pltpu.sync_copy(ubuf, u_hbm.at[pl.ds(base, CH)])
            return 0

        lax.fori_loop(0, per_tile, chunk, 0)

    return u_kernel


def _make_render_kernel(B, H, W, KN, N, NB):
    HW = H * W
    NCOV = N - NB
    CH = 32
    imp_chunks = (B * NB) // CH
    cov_chunks = (B * NCOV) // CH
    imp_per_tile = imp_chunks // _NW
    cov_per_tile = cov_chunks // _NW
    mesh = plsc.VectorSubcoreMesh(core_axis_name="c", subcore_axis_name="s")

    @functools.partial(
        pl.kernel, mesh=mesh,
        out_type=(
            jax.ShapeDtypeStruct((B * NB,), jnp.float32),
            jax.ShapeDtypeStruct((B * NB,), jnp.float32),
            jax.ShapeDtypeStruct((B * _NC * N,), jnp.float32),
        ),
        scratch_types=[
            pltpu.VMEM((CH,), jnp.int32),
            pltpu.VMEM((CH,), jnp.int32),
            pltpu.VMEM((CH,), jnp.float32),
            pltpu.VMEM((CH,), jnp.float32),
            pltpu.VMEM((24, _L), jnp.float32),
            pltpu.VMEM((3, CH), jnp.float32),
            pltpu.SemaphoreType.DMA,
            pltpu.SemaphoreType.DMA,
        ],
    )
    def render_kernel(idx_hbm, ox_hbm, oy_hbm, cx_hbm, cy_hbm, y_hbm,
                      ptsx_hbm, ptsy_hbm, rend_hbm,
                      ibuf, givbuf, pxbuf, pybuf, gbuf, rbuf, sem, sem2):
        wid = lax.axis_index("s") * _SC_CORES + lax.axis_index("c")

        def sample(b, tglob):
            copies = []
            geom = []
            for j in range(CH // _L):
                px = pxbuf[pl.ds(j * _L, _L)]
                py = pybuf[pl.ds(j * _L, _L)]
                x0f, x0i, y0f, y0i, wx0, wx1, wy0, wy1 = _grid_coords(
                    px, py, H, W)
                corners = [
                    (x0f, y0f, x0i, y0i),
                    (x0f + 1.0, y0f, x0i + 1, y0i),
                    (x0f, y0f + 1.0, x0i, y0i + 1),
                    (x0f + 1.0, y0f + 1.0, x0i + 1, y0i + 1),
                ]
                ws = [wy0 * wx0, wy0 * wx1, wy1 * wx0, wy1 * wx1]
                vms = []
                for q, (xf, yf, xi, yi) in enumerate(corners):
                    lin, vm = _corner(xf, yf, xi, yi, H, W)
                    vms.append(vm)
                    for ch in range(3):
                        gidx = (b * 3 + ch) * HW + lin
                        copies.append(pltpu.async_copy(
                            y_hbm.at[gidx], gbuf.at[12 * j + 3 * q + ch],
                            sem if ch != 1 else sem2))
                geom.append((ws, vms))
            for cp in copies:
                cp.wait()
            for j in range(CH // _L):
                ws, vms = geom[j]
                for ch in range(3):
                    v = [gbuf[12 * j + 3 * q + ch] * vms[q]
                         for q in range(4)]
                    r = ((v[0] * ws[0] + v[1] * ws[1]) + v[2] * ws[2]) \
                        + v[3] * ws[3]
                    rbuf[ch, pl.ds(j * _L, _L)] = jnp.maximum(r, 0.0)
            for ch in range(3):
                pltpu.sync_copy(
                    rbuf.at[ch],
                    rend_hbm.at[pl.ds((b * 3 + ch) * N + tglob, CH)])
            return 0

        def imp_chunk(i, _):
            cid = wid * imp_per_tile + i
            base = cid * CH
            b = base // NB
            t = base - b * NB
            pltpu.sync_copy(idx_hbm.at[pl.ds(base, CH)], ibuf)
            for j in range(CH // _L):
                sl = pl.ds(j * _L, _L)
                givbuf[sl] = ibuf[sl] + b * KN
            cpx = pltpu.async_copy(ox_hbm.at[givbuf], pxbuf, sem)
            cpy = pltpu.async_copy(oy_hbm.at[givbuf], pybuf, sem2)
            cpx.wait()
            cpy.wait()
            pltpu.sync_copy(pxbuf, ptsx_hbm.at[pl.ds(base, CH)])
            pltpu.sync_copy(pybuf, ptsy_hbm.at[pl.ds(base, CH)])
            return sample(b, t)

        def cov_chunk(i, _):
            cid = wid * cov_per_tile + i
            base = cid * CH
            b = base // NCOV
            t = base - b * NCOV
            pltpu.sync_copy(cx_hbm.at[pl.ds(base, CH)], pxbuf)
            pltpu.sync_copy(cy_hbm.at[pl.ds(base, CH)], pybuf)
            return sample(b, NB + t)

        lax.fori_loop(0, imp_per_tile, imp_chunk, 0)
        lax.fori_loop(0, cov_per_tile, cov_chunk, 0)

    return render_kernel


def kernel(x, res, out, W_mlp):
    B, C, H, W = out.shape
    HW = H * W
    N = HW // 16
    KN = _KS * N
    NB = int(_BETA * N)

    key = jax.random.key(42)
    k1, k2 = jax.random.split(key)
    over = jax.random.uniform(k1, (B, KN, 2), dtype=x.dtype)
    coverage = jax.random.uniform(k2, (B, N - NB, 2), dtype=x.dtype)

    x2 = x.reshape(B, _EMBED, HW)
    o2 = out.reshape(B, C, HW)
    r2 = res.reshape(B, C, HW)
    s0, s1 = _smap_maps(o2, r2, B, HW)
    y = _ymap_map(x2, o2, W_mlp, B, HW)

    ox = over[:, :, 0].reshape(B * KN)
    oy = over[:, :, 1].reshape(B * KN)
    cx = coverage[:, :, 0].reshape(B * (N - NB))
    cy = coverage[:, :, 1].reshape(B * (N - NB))

    u_kernel = _make_u_kernel(B, H, W, KN)
    u = u_kernel(ox, oy, s0.reshape(B * HW), s1.reshape(B * HW))
    u = u.reshape(B, KN)

    _, idx = lax.top_k(u, NB)

    render = _make_render_kernel(B, H, W, KN, N, NB)
    ptsx, ptsy, rend = render(
        idx.reshape(B * NB).astype(jnp.int32),
        ox, oy, cx, cy,
        y.reshape(B * C * HW),
    )
    pts_imp = jnp.stack([ptsx, ptsy], axis=-1).reshape(B, NB, 2)
    points = jnp.concatenate([pts_imp, coverage], axis=1)
    rend = rend.reshape(B, C, N)
    return rend, points


# speedup vs baseline: 2.6440x; 1.0205x over previous
import functools

import jax
import jax.numpy as jnp
from jax import lax
from jax.experimental import pallas as pl
from jax.experimental.pallas import tpu as pltpu
from jax.experimental.pallas import tpu_sc as plsc

_EMBED = 96
_NC = 3
_KS = 3
_BETA = 0.75

_SC_CORES = 2
_SC_SUBCORES = 16
_NW = _SC_CORES * _SC_SUBCORES
_L = 16


def _smap_body(o_ref, r_ref, s_ref):
    o = o_ref[0]
    d = o - r_ref[0]
    d = d * d
    d0, d1, d2 = d[0:1], d[1:2], d[2:3]
    hi = jnp.maximum(d0, d1)
    lo = jnp.minimum(d0, d1)
    s0 = jnp.maximum(hi, d2)
    s1 = jnp.minimum(hi, jnp.maximum(lo, d2))
    s_ref[0] = jnp.concatenate([s0, s1], axis=0)


def _smap_maps(o2, r2, B, HW):
    T = 4096
    grid = (B, HW // T)
    return pl.pallas_call(
        _smap_body,
        grid=grid,
        in_specs=[
            pl.BlockSpec((1, _NC, T), lambda b, t: (b, 0, t)),
            pl.BlockSpec((1, _NC, T), lambda b, t: (b, 0, t)),
        ],
        out_specs=pl.BlockSpec((1, 2, T), lambda b, t: (b, 0, t)),
        out_shape=jax.ShapeDtypeStruct((B, 2, HW), jnp.float32),
    )(o2, r2)


def _ymap_body(w_ref, x_ref, o_ref, y_ref):
    o = o_ref[0]
    w = w_ref[...]
    f = x_ref[0]
    y = lax.dot_general(w[:, :_NC], o, (((1,), (0,)), ((), ())),
                        preferred_element_type=jnp.float32)
    y = y + lax.dot_general(w[:, _NC:], f, (((1,), (0,)), ((), ())),
                            preferred_element_type=jnp.float32)
    y_ref[0] = y


def _ymap_map(x2, o2, w_mlp, B, HW):
    T = 2048
    grid = (B, HW // T)
    return pl.pallas_call(
        _ymap_body,
        grid=grid,
        in_specs=[
            pl.BlockSpec((_NC, _EMBED + _NC), lambda b, t: (0, 0)),
            pl.BlockSpec((1, _EMBED, T), lambda b, t: (b, 0, t)),
            pl.BlockSpec((1, _NC, T), lambda b, t: (b, 0, t)),
        ],
        out_specs=pl.BlockSpec((1, _NC, T), lambda b, t: (b, 0, t)),
        out_shape=jax.ShapeDtypeStruct((B, _NC, HW), jnp.float32),
    )(w_mlp, x2, o2)


def _grid_coords(px, py, H, W):
    gridx = 2.0 * px - 1.0
    gridy = 2.0 * py - 1.0
    gx = ((gridx + 1.0) * W - 1.0) / 2.0
    gy = ((gridy + 1.0) * H - 1.0) / 2.0

    def _floor(g):
        ti = g.astype(jnp.int32)
        tf = ti.astype(jnp.float32)
        adj = g < tf
        tf = jnp.where(adj, tf - 1.0, tf)
        ti = jnp.where(adj, ti - 1, ti)
        return tf, ti

    x0f, x0i = _floor(gx)
    y0f, y0i = _floor(gy)
    wx1 = gx - x0f
    wx0 = 1.0 - wx1
    wy1 = gy - y0f
    wy0 = 1.0 - wy1
    return x0f, x0i, y0f, y0i, wx0, wx1, wy0, wy1


def _corner(xf, yf, xi, yi, H, W):
    valid = (xf >= 0.0) & (xf <= W - 1.0) & (yf >= 0.0) & (yf <= H - 1.0)
    vm = jnp.where(valid, 1.0, 0.0)
    ix = jnp.clip(xi, 0, W - 1)
    iy = jnp.clip(yi, 0, H - 1)
    return iy * W + ix, vm


def _make_u_kernel(B, H, W, KN):
    HW = H * W
    total = B * KN
    CH = 96
    NV = CH // _L
    nchunks = total // CH
    per_tile = nchunks // _NW
    mesh = plsc.VectorSubcoreMesh(core_axis_name="c", subcore_axis_name="s")

    @functools.partial(
        pl.kernel, mesh=mesh,
        out_type=jax.ShapeDtypeStruct((total,), jnp.float32),
        scratch_types=[
            pltpu.VMEM((CH,), jnp.float32),
            pltpu.VMEM((CH,), jnp.float32),
            pltpu.VMEM((4 * NV, 2 * _L), jnp.int32),
            pltpu.VMEM((4 * NV, 2 * _L), jnp.float32),
            pltpu.VMEM((CH,), jnp.float32),
            pltpu.SemaphoreType.DMA,
            pltpu.SemaphoreType.DMA,
        ],
    )
    def u_kernel(overx_hbm, overy_hbm, smap_hbm, u_hbm,
                 cbufx, cbufy, ibuf, gbuf, ubuf, sem, sem2):
        wid = lax.axis_index("s") * _SC_CORES + lax.axis_index("c")

        def chunk(i, _):
            cid = wid * per_tile + i
            base = cid * CH
            b = base // KN
            pltpu.sync_copy(overx_hbm.at[pl.ds(base, CH)], cbufx)
            pltpu.sync_copy(overy_hbm.at[pl.ds(base, CH)], cbufy)
            geom = []
            for j in range(NV):
                px = cbufx[pl.ds(j * _L, _L)]
                py = cbufy[pl.ds(j * _L, _L)]
                x0f, x0i, y0f, y0i, wx0, wx1, wy0, wy1 = _grid_coords(
                    px, py, H, W)
                corners = [
                    (x0f, y0f, x0i, y0i),
                    (x0f + 1.0, y0f, x0i + 1, y0i),
                    (x0f, y0f + 1.0, x0i, y0i + 1),
                    (x0f + 1.0, y0f + 1.0, x0i + 1, y0i + 1),
                ]
                ws = [wy0 * wx0, wy0 * wx1, wy1 * wx0, wy1 * wx1]
                vms = []
                for q, (xf, yf, xi, yi) in enumerate(corners):
                    lin, vm = _corner(xf, yf, xi, yi, H, W)
                    gidx = (2 * b) * HW + lin
                    vms.append(vm)
                    slot = 4 * j + q
                    ibuf[slot, pl.ds(0, _L)] = gidx
                    ibuf[slot, pl.ds(_L, _L)] = gidx + HW
                geom.append((ws, vms))
            copies = []
            for j in range(NV):
                for q in range(4):
                    slot = 4 * j + q
                    copies.append(pltpu.async_copy(
                        smap_hbm.at[ibuf.at[slot]], gbuf.at[slot],
                        sem if q % 2 == 0 else sem2))
            for cp in copies:
                cp.wait()
            for j in range(NV):
                ws, vms = geom[j]
                og = []
                for p in range(2):
                    v = [gbuf[4 * j + q, pl.ds(p * _L, _L)] * vms[q]
                         for q in range(4)]
                    og.append(((v[0] * ws[0] + v[1] * ws[1]) + v[2] * ws[2])
                              + v[3] * ws[3])
                u = -1.0 * (og[0] - og[1])
                ubuf[pl.ds(j * _L, _L)] = u
            pltpu.sync_copy(ubuf, u_hbm.at[pl.ds(base, CH)])
            return 0

        lax.fori_loop(0, per_tile, chunk, 0)

    return u_kernel


def _make_render_kernel(B, H, W, KN, N, NB):
    HW = H * W
    NCOV = N - NB
    CH = 48
    NV = CH // _L
    imp_chunks = (B * NB) // CH
    cov_chunks = (B * NCOV) // CH
    imp_per_tile = imp_chunks // _NW
    cov_per_tile = cov_chunks // _NW
    mesh = plsc.VectorSubcoreMesh(core_axis_name="c", subcore_axis_name="s")

    @functools.partial(
        pl.kernel, mesh=mesh,
        out_type=(
            jax.ShapeDtypeStruct((B * NB,), jnp.float32),
            jax.ShapeDtypeStruct((B * NB,), jnp.float32),
            jax.ShapeDtypeStruct((B * _NC * N,), jnp.float32),
        ),
        scratch_types=[
            pltpu.VMEM((CH,), jnp.int32),
            pltpu.VMEM((CH,), jnp.int32),
            pltpu.VMEM((CH,), jnp.float32),
            pltpu.VMEM((CH,), jnp.float32),
            pltpu.VMEM((4 * NV, 3 * _L), jnp.int32),
            pltpu.VMEM((4 * NV, 3 * _L), jnp.float32),
            pltpu.VMEM((3, CH), jnp.float32),
            pltpu.SemaphoreType.DMA,
            pltpu.SemaphoreType.DMA,
        ],
    )
    def render_kernel(idx_hbm, ox_hbm, oy_hbm, cx_hbm, cy_hbm, y_hbm,
                      ptsx_hbm, ptsy_hbm, rend_hbm,
                      ibuf, givbuf, pxbuf, pybuf, gibuf, gbuf, rbuf,
                      sem, sem2):
        wid = lax.axis_index("s") * _SC_CORES + lax.axis_index("c")

        def sample(b, tglob):
            geom = []
            for j in range(NV):
                px = pxbuf[pl.ds(j * _L, _L)]
                py = pybuf[pl.ds(j * _L, _L)]
                x0f, x0i, y0f, y0i, wx0, wx1, wy0, wy1 = _grid_coords(
                    px, py, H, W)
                corners = [
                    (x0f, y0f, x0i, y0i),
                    (x0f + 1.0, y0f, x0i + 1, y0i),
                    (x0f, y0f + 1.0, x0i, y0i + 1),
                    (x0f + 1.0, y0f + 1.0, x0i + 1, y0i + 1),
                ]
                ws = [wy0 * wx0, wy0 * wx1, wy1 * wx0, wy1 * wx1]
                vms = []
                for q, (xf, yf, xi, yi) in enumerate(corners):
                    lin, vm = _corner(xf, yf, xi, yi, H, W)
                    gidx = (b * 3) * HW + lin
                    vms.append(vm)
                    slot = 4 * j + q
                    ibase = gidx
                    gibuf[slot, pl.ds(0, _L)] = ibase
                    gibuf[slot, pl.ds(_L, _L)] = ibase + HW
                    gibuf[slot, pl.ds(2 * _L, _L)] = ibase + 2 * HW
                geom.append((ws, vms))
            copies = []
            for j in range(NV):
                for q in range(4):
                    slot = 4 * j + q
                    copies.append(pltpu.async_copy(
                        y_hbm.at[gibuf.at[slot]], gbuf.at[slot],
                        sem if q % 2 == 0 else sem2))
            for cp in copies:
                cp.wait()
            for j in range(NV):
                ws, vms = geom[j]
                for ch in range(3):
                    v = [gbuf[4 * j + q, pl.ds(ch * _L, _L)] * vms[q]
                         for q in range(4)]
                    r = ((v[0] * ws[0] + v[1] * ws[1]) + v[2] * ws[2]) \
                        + v[3] * ws[3]
                    rbuf[ch, pl.ds(j * _L, _L)] = jnp.maximum(r, 0.0)
            for ch in range(3):
                pltpu.sync_copy(
                    rbuf.at[ch],
                    rend_hbm.at[pl.ds((b * 3 + ch) * N + tglob, CH)])
            return 0

        def imp_chunk(i, _):
            cid = wid * imp_per_tile + i
            base = cid * CH
            b = base // NB
            t = base - b * NB
            pltpu.sync_copy(idx_hbm.at[pl.ds(base, CH)], ibuf)
            for j in range(CH // _L):
                sl = pl.ds(j * _L, _L)
                givbuf[sl] = ibuf[sl] + b * KN
            cpx = pltpu.async_copy(ox_hbm.at[givbuf], pxbuf, sem)
            cpy = pltpu.async_copy(oy_hbm.at[givbuf], pybuf, sem2)
            cpx.wait()
            cpy.wait()
            pltpu.sync_copy(pxbuf, ptsx_hbm.at[pl.ds(base, CH)])
            pltpu.sync_copy(pybuf, ptsy_hbm.at[pl.ds(base, CH)])
            return sample(b, t)

        def cov_chunk(i, _):
            cid = wid * cov_per_tile + i
            base = cid * CH
            b = base // NCOV
            t = base - b * NCOV
            pltpu.sync_copy(cx_hbm.at[pl.ds(base, CH)], pxbuf)
            pltpu.sync_copy(cy_hbm.at[pl.ds(base, CH)], pybuf)
            return sample(b, NB + t)

        lax.fori_loop(0, imp_per_tile, imp_chunk, 0)
        lax.fori_loop(0, cov_per_tile, cov_chunk, 0)

    return render_kernel


def kernel(x, res, out, W_mlp):
    B, C, H, W = out.shape
    HW = H * W
    N = HW // 16
    KN = _KS * N
    NB = int(_BETA * N)

    key = jax.random.key(42)
    k1, k2 = jax.random.split(key)
    over = jax.random.uniform(k1, (B, KN, 2), dtype=x.dtype)
    coverage = jax.random.uniform(k2, (B, N - NB, 2), dtype=x.dtype)

    x2 = x.reshape(B, _EMBED, HW)
    o2 = out.reshape(B, C, HW)
    r2 = res.reshape(B, C, HW)
    smap = _smap_maps(o2, r2, B, HW)
    y = _ymap_map(x2, o2, W_mlp, B, HW)

    ox = over[:, :, 0].reshape(B * KN)
    oy = over[:, :, 1].reshape(B * KN)
    cx = coverage[:, :, 0].reshape(B * (N - NB))
    cy = coverage[:, :, 1].reshape(B * (N - NB))

    u_kernel = _make_u_kernel(B, H, W, KN)
    u = u_kernel(ox, oy, smap.reshape(B * 2 * HW))
    u = u.reshape(B, KN)

    _, idx = lax.top_k(u, NB)

    render = _make_render_kernel(B, H, W, KN, N, NB)
    ptsx, ptsy, rend = render(
        idx.reshape(B * NB).astype(jnp.int32),
        ox, oy, cx, cy,
        y.reshape(B * C * HW),
    )
    pts_imp = jnp.stack([ptsx, ptsy], axis=-1).reshape(B, NB, 2)
    points = jnp.concatenate([pts_imp, coverage], axis=1)
    rend = rend.reshape(B, C, N)
    return rend, points


# speedup vs baseline: 2.9228x; 1.1055x over previous
import functools

import jax
import jax.numpy as jnp
from jax import lax
from jax.experimental import pallas as pl
from jax.experimental.pallas import tpu as pltpu
from jax.experimental.pallas import tpu_sc as plsc

_EMBED = 96
_NC = 3
_KS = 3
_BETA = 0.75

_SC_CORES = 2
_SC_SUBCORES = 16
_NW = _SC_CORES * _SC_SUBCORES
_L = 16


def _smap_body(o_ref, r_ref, s_ref):
    o = o_ref[0]
    d = o - r_ref[0]
    d = d * d
    d0, d1, d2 = d[0:1], d[1:2], d[2:3]
    hi = jnp.maximum(d0, d1)
    lo = jnp.minimum(d0, d1)
    s0 = jnp.maximum(hi, d2)
    s1 = jnp.minimum(hi, jnp.maximum(lo, d2))
    s_ref[0] = jnp.concatenate([s0, s1], axis=0)


def _smap_maps(o2, r2, B, HW):
    T = 4096
    grid = (B, HW // T)
    return pl.pallas_call(
        _smap_body,
        grid=grid,
        in_specs=[
            pl.BlockSpec((1, _NC, T), lambda b, t: (b, 0, t)),
            pl.BlockSpec((1, _NC, T), lambda b, t: (b, 0, t)),
        ],
        out_specs=pl.BlockSpec((1, 2, T), lambda b, t: (b, 0, t)),
        out_shape=jax.ShapeDtypeStruct((B, 2, HW), jnp.float32),
    )(o2, r2)


def _ymap_body(w_ref, x_ref, o_ref, y_ref):
    o = o_ref[0]
    w = w_ref[...]
    f = x_ref[0]
    y = lax.dot_general(w[:, :_NC], o, (((1,), (0,)), ((), ())),
                        preferred_element_type=jnp.float32)
    y = y + lax.dot_general(w[:, _NC:], f, (((1,), (0,)), ((), ())),
                            preferred_element_type=jnp.float32)
    y_ref[0] = y


def _ymap_map(x2, o2, w_mlp, B, HW):
    T = 2048
    grid = (B, HW // T)
    return pl.pallas_call(
        _ymap_body,
        grid=grid,
        in_specs=[
            pl.BlockSpec((_NC, _EMBED + _NC), lambda b, t: (0, 0)),
            pl.BlockSpec((1, _EMBED, T), lambda b, t: (b, 0, t)),
            pl.BlockSpec((1, _NC, T), lambda b, t: (b, 0, t)),
        ],
        out_specs=pl.BlockSpec((1, _NC, T), lambda b, t: (b, 0, t)),
        out_shape=jax.ShapeDtypeStruct((B, _NC, HW), jnp.float32),
    )(w_mlp, x2, o2)


def _grid_coords(px, py, H, W):
    gridx = 2.0 * px - 1.0
    gridy = 2.0 * py - 1.0
    gx = ((gridx + 1.0) * W - 1.0) / 2.0
    gy = ((gridy + 1.0) * H - 1.0) / 2.0

    def _floor(g):
        ti = g.astype(jnp.int32)
        tf = ti.astype(jnp.float32)
        adj = g < tf
        tf = jnp.where(adj, tf - 1.0, tf)
        ti = jnp.where(adj, ti - 1, ti)
        return tf, ti

    x0f, x0i = _floor(gx)
    y0f, y0i = _floor(gy)
    wx1 = gx - x0f
    wx0 = 1.0 - wx1
    wy1 = gy - y0f
    wy0 = 1.0 - wy1
    return x0f, x0i, y0f, y0i, wx0, wx1, wy0, wy1


def _corner(xf, yf, xi, yi, H, W):
    valid = (xf >= 0.0) & (xf <= W - 1.0) & (yf >= 0.0) & (yf <= H - 1.0)
    vm = jnp.where(valid, 1.0, 0.0)
    ix = jnp.clip(xi, 0, W - 1)
    iy = jnp.clip(yi, 0, H - 1)
    return iy * W + ix, vm


def _make_u_kernel(B, H, W, KN):
    HW = H * W
    total = B * KN
    CH = 96
    NV = CH // _L
    nchunks = total // CH
    per_tile = nchunks // _NW
    mesh = plsc.VectorSubcoreMesh(core_axis_name="c", subcore_axis_name="s")

    @functools.partial(
        pl.kernel, mesh=mesh,
        out_type=jax.ShapeDtypeStruct((total,), jnp.float32),
        scratch_types=[
            pltpu.VMEM((CH,), jnp.float32),
            pltpu.VMEM((CH,), jnp.float32),
            pltpu.VMEM((4 * NV, 2 * _L), jnp.int32),
            pltpu.VMEM((4 * NV, 2 * _L), jnp.float32),
            pltpu.VMEM((CH,), jnp.float32),
            pltpu.SemaphoreType.DMA,
            pltpu.SemaphoreType.DMA,
        ],
    )
    def u_kernel(overx_hbm, overy_hbm, smap_hbm, u_hbm,
                 cbufx, cbufy, ibuf, gbuf, ubuf, sem, sem2):
        wid = lax.axis_index("s") * _SC_CORES + lax.axis_index("c")

        def chunk(i, _):
            cid = wid * per_tile + i
            base = cid * CH
            b = base // KN
            pltpu.sync_copy(overx_hbm.at[pl.ds(base, CH)], cbufx)
            pltpu.sync_copy(overy_hbm.at[pl.ds(base, CH)], cbufy)
            geom = []
            for j in range(NV):
                px = cbufx[pl.ds(j * _L, _L)]
                py = cbufy[pl.ds(j * _L, _L)]
                x0f, x0i, y0f, y0i, wx0, wx1, wy0, wy1 = _grid_coords(
                    px, py, H, W)
                corners = [
                    (x0f, y0f, x0i, y0i),
                    (x0f + 1.0, y0f, x0i + 1, y0i),
                    (x0f, y0f + 1.0, x0i, y0i + 1),
                    (x0f + 1.0, y0f + 1.0, x0i + 1, y0i + 1),
                ]
                ws = [wy0 * wx0, wy0 * wx1, wy1 * wx0, wy1 * wx1]
                vms = []
                for q, (xf, yf, xi, yi) in enumerate(corners):
                    lin, vm = _corner(xf, yf, xi, yi, H, W)
                    gidx = (2 * b) * HW + lin
                    vms.append(vm)
                    slot = 4 * j + q
                    ibuf[slot, pl.ds(0, _L)] = gidx
                    ibuf[slot, pl.ds(_L, _L)] = gidx + HW
                geom.append((ws, vms))
            copies = []
            for j in range(NV):
                for q in range(4):
                    slot = 4 * j + q
                    copies.append(pltpu.async_copy(
                        smap_hbm.at[ibuf.at[slot]], gbuf.at[slot],
                        sem if q % 2 == 0 else sem2))
            for cp in copies:
                cp.wait()
            for j in range(NV):
                ws, vms = geom[j]
                og = []
                for p in range(2):
                    v = [gbuf[4 * j + q, pl.ds(p * _L, _L)] * vms[q]
                         for q in range(4)]
                    og.append(((v[0] * ws[0] + v[1] * ws[1]) + v[2] * ws[2])
                              + v[3] * ws[3])
                u = -1.0 * (og[0] - og[1])
                ubuf[pl.ds(j * _L, _L)] = u
            pltpu.sync_copy(ubuf, u_hbm.at[pl.ds(base, CH)])
            return 0

        lax.fori_loop(0, per_tile, chunk, 0)

    return u_kernel


def _make_render_kernel(B, H, W, KN, N, NB):
    HW = H * W
    NCOV = N - NB
    CH = 48
    NV = CH // _L
    assert (B * NB) % CH == 0 and (B * NCOV) % CH == 0
    imp_chunks = (B * NB) // CH
    cov_chunks = (B * NCOV) // CH
    imp_per_tile = -(-imp_chunks // _NW)
    cov_per_tile = -(-cov_chunks // _NW)
    mesh = plsc.VectorSubcoreMesh(core_axis_name="c", subcore_axis_name="s")

    @functools.partial(
        pl.kernel, mesh=mesh,
        out_type=(
            jax.ShapeDtypeStruct((B * NB,), jnp.float32),
            jax.ShapeDtypeStruct((B * NB,), jnp.float32),
            jax.ShapeDtypeStruct((B * _NC * N,), jnp.float32),
        ),
        scratch_types=[
            pltpu.VMEM((CH,), jnp.int32),
            pltpu.VMEM((CH,), jnp.int32),
            pltpu.VMEM((CH,), jnp.float32),
            pltpu.VMEM((CH,), jnp.float32),
            pltpu.VMEM((4 * NV, 3 * _L), jnp.int32),
            pltpu.VMEM((4 * NV, 3 * _L), jnp.float32),
            pltpu.VMEM((3, CH), jnp.float32),
            pltpu.SemaphoreType.DMA,
            pltpu.SemaphoreType.DMA,
        ],
    )
    def render_kernel(idx_hbm, ox_hbm, oy_hbm, cx_hbm, cy_hbm, y_hbm,
                      ptsx_hbm, ptsy_hbm, rend_hbm,
                      ibuf, givbuf, pxbuf, pybuf, gibuf, gbuf, rbuf,
                      sem, sem2):
        wid = lax.axis_index("s") * _SC_CORES + lax.axis_index("c")

        def sample(b, tglob):
            geom = []
            for j in range(NV):
                px = pxbuf[pl.ds(j * _L, _L)]
                py = pybuf[pl.ds(j * _L, _L)]
                x0f, x0i, y0f, y0i, wx0, wx1, wy0, wy1 = _grid_coords(
                    px, py, H, W)
                corners = [
                    (x0f, y0f, x0i, y0i),
                    (x0f + 1.0, y0f, x0i + 1, y0i),
                    (x0f, y0f + 1.0, x0i, y0i + 1),
                    (x0f + 1.0, y0f + 1.0, x0i + 1, y0i + 1),
                ]
                ws = [wy0 * wx0, wy0 * wx1, wy1 * wx0, wy1 * wx1]
                vms = []
                for q, (xf, yf, xi, yi) in enumerate(corners):
                    lin, vm = _corner(xf, yf, xi, yi, H, W)
                    gidx = (b * 3) * HW + lin
                    vms.append(vm)
                    slot = 4 * j + q
                    ibase = gidx
                    gibuf[slot, pl.ds(0, _L)] = ibase
                    gibuf[slot, pl.ds(_L, _L)] = ibase + HW
                    gibuf[slot, pl.ds(2 * _L, _L)] = ibase + 2 * HW
                geom.append((ws, vms))
            copies = []
            for j in range(NV):
                for q in range(4):
                    slot = 4 * j + q
                    copies.append(pltpu.async_copy(
                        y_hbm.at[gibuf.at[slot]], gbuf.at[slot],
                        sem if q % 2 == 0 else sem2))
            for cp in copies:
                cp.wait()
            for j in range(NV):
                ws, vms = geom[j]
                for ch in range(3):
                    v = [gbuf[4 * j + q, pl.ds(ch * _L, _L)] * vms[q]
                         for q in range(4)]
                    r = ((v[0] * ws[0] + v[1] * ws[1]) + v[2] * ws[2]) \
                        + v[3] * ws[3]
                    rbuf[ch, pl.ds(j * _L, _L)] = jnp.maximum(r, 0.0)
            for ch in range(3):
                pltpu.sync_copy(
                    rbuf.at[ch],
                    rend_hbm.at[pl.ds((b * 3 + ch) * N + tglob, CH)])
            return 0

        def imp_chunk(i, _):
            cid = i * _NW + wid

            @pl.when(cid < imp_chunks)
            def _():
                base = cid * CH
                b = base // NB
                t = base - b * NB
                pltpu.sync_copy(idx_hbm.at[pl.ds(base, CH)], ibuf)
                for j in range(CH // _L):
                    sl = pl.ds(j * _L, _L)
                    givbuf[sl] = ibuf[sl] + b * KN
                cpx = pltpu.async_copy(ox_hbm.at[givbuf], pxbuf, sem)
                cpy = pltpu.async_copy(oy_hbm.at[givbuf], pybuf, sem2)
                cpx.wait()
                cpy.wait()
                pltpu.sync_copy(pxbuf, ptsx_hbm.at[pl.ds(base, CH)])
                pltpu.sync_copy(pybuf, ptsy_hbm.at[pl.ds(base, CH)])
                sample(b, t)
            return 0

        def cov_chunk(i, _):
            cid = i * _NW + wid

            @pl.when(cid < cov_chunks)
            def _():
                base = cid * CH
                b = base // NCOV
                t = base - b * NCOV
                pltpu.sync_copy(cx_hbm.at[pl.ds(base, CH)], pxbuf)
                pltpu.sync_copy(cy_hbm.at[pl.ds(base, CH)], pybuf)
                sample(b, NB + t)
            return 0

        lax.fori_loop(0, imp_per_tile, imp_chunk, 0)
        lax.fori_loop(0, cov_per_tile, cov_chunk, 0)

    return render_kernel


def kernel(x, res, out, W_mlp):
    B, C, H, W = out.shape
    HW = H * W
    N = HW // 16
    KN = _KS * N
    NB = int(_BETA * N)

    key = jax.random.key(42)
    k1, k2 = jax.random.split(key)
    over = jax.random.uniform(k1, (B, KN, 2), dtype=x.dtype)
    coverage = jax.random.uniform(k2, (B, N - NB, 2), dtype=x.dtype)

    x2 = x.reshape(B, _EMBED, HW)
    o2 = out.reshape(B, C, HW)
    r2 = res.reshape(B, C, HW)
    smap = _smap_maps(o2, r2, B, HW)
    y = _ymap_map(x2, o2, W_mlp, B, HW)

    ox = over[:, :, 0]
    oy = over[:, :, 1]
    cx = coverage[:, :, 0]
    cy = coverage[:, :, 1]
    smap_f = smap.reshape(B, 2 * HW)
    y_f = y.reshape(B, C * HW)

    u_kernel = _make_u_kernel(1, H, W, KN)
    render = _make_render_kernel(1, H, W, KN, N, NB)
    ptsx_l, ptsy_l, rend_l = [], [], []
    for b in range(B):
        u_b = u_kernel(ox[b], oy[b], smap_f[b])
        _, idx_b = lax.top_k(u_b, NB)
        px_b, py_b, rend_b = render(
            idx_b.astype(jnp.int32), ox[b], oy[b], cx[b], cy[b], y_f[b])
        ptsx_l.append(px_b)
        ptsy_l.append(py_b)
        rend_l.append(rend_b)

    ptsx = jnp.stack(ptsx_l)
    ptsy = jnp.stack(ptsy_l)
    pts_imp = jnp.stack([ptsx, ptsy], axis=-1)
    points = jnp.concatenate([pts_imp, coverage], axis=1)
    rend = jnp.stack(rend_l).reshape(B, C, N)
    return rend, points
